# baseline (speedup 1.0000x reference)
import functools
import math

import jax
import jax.numpy as jnp
from jax import lax
from jax.experimental import pallas as pl
from jax.experimental.pallas import tpu as pltpu
from jax.experimental.pallas import tpu_sc as plsc

_B, _N, _K, _ET = 4, 256, 128, 1024
_ROWS = _B * _N
_HALF = _ROWS // 2
_NC, _NS = 2, 16
_NW = _NC * _NS
_LANES = 16
_RB = 128


def _sc_affine(x2, et2, mul_flat, bias_flat, base, nrows):
    rpw = nrows // _NW
    nsl = _N // _LANES

    def body(x_hbm, et_hbm, mul_hbm, bias_hbm, out_hbm,
             xv, etv, mulv, biasv, outv):
        wid = lax.axis_index("s") * _NC + lax.axis_index("c")
        src = base + wid * rpw
        dst = wid * rpw
        pltpu.sync_copy(mul_hbm, mulv)
        pltpu.sync_copy(bias_hbm, biasv)
        pltpu.sync_copy(x_hbm.at[pl.ds(src, rpw)], xv)
        pltpu.sync_copy(et_hbm.at[pl.ds(src, rpw)], etv)

        @plsc.parallel_loop(0, rpw * nsl, unroll=8)
        def _(i):
            r = i // nsl
            sl = pl.ds((i % nsl) * _LANES, _LANES)
            idx = etv[r, sl]
            m = plsc.load_gather(mulv, [idx])
            b = plsc.load_gather(biasv, [idx])
            outv[r, sl] = m * xv[r, sl] + b

        pltpu.sync_copy(outv, out_hbm.at[pl.ds(dst, rpw)])

    mesh = plsc.VectorSubcoreMesh(core_axis_name="c", subcore_axis_name="s")
    kern = functools.partial(
        pl.kernel,
        mesh=mesh,
        compiler_params=pltpu.CompilerParams(needs_layout_passes=False),
        out_type=jax.ShapeDtypeStruct((nrows, _N), jnp.float32),
        scratch_types=[
            pltpu.VMEM((rpw, _N), jnp.float32),
            pltpu.VMEM((rpw, _N), jnp.int32),
            pltpu.VMEM((_ET,), jnp.float32),
            pltpu.VMEM((_ET,), jnp.float32),
            pltpu.VMEM((rpw, _N), jnp.float32),
        ],
    )(body)
    return kern(x2, et2, mul_flat, bias_flat)


def _gauss_body(xe_ref, mean_ref, std_ref, out_ref):
    mean = mean_ref[...].reshape(1, 1, _K)
    s = jnp.abs(std_ref[...]).reshape(1, 1, _K) + 1e-5
    g = math.sqrt(0.5 * math.log2(math.e)) / s
    log2coef = -jnp.log2(s) - math.log2(math.sqrt(2.0 * math.pi))
    xv = xe_ref[...]
    t = (xv[:, :, None] - mean) * g
    out_ref[...] = jnp.exp2(log2coef - t * t)


def _tc_gauss_first(xe_ref, mean_ref, std_ref, out_ref):
    _gauss_body(xe_ref, mean_ref, std_ref, out_ref)


def _tc_gauss_second(xe_ref, mean_ref, std_ref, buf_ref, out_ref):
    del buf_ref
    _gauss_body(xe_ref, mean_ref, std_ref, out_ref)


def _tc_gauss(xe1, xe2, means, stds):
    nblk = _HALF // _RB
    small = pl.BlockSpec((1, _K), lambda i: (0, 0))
    xspec = pl.BlockSpec((_RB, _N), lambda i: (i, 0))
    out_shape = jax.ShapeDtypeStruct((_ROWS, _N, _K), jnp.float32)
    buf = pl.pallas_call(
        _tc_gauss_first,
        grid=(nblk,),
        in_specs=[xspec, small, small],
        out_specs=pl.BlockSpec((_RB, _N, _K), lambda i: (i, 0, 0)),
        out_shape=out_shape,
    )(xe1, means, stds)
    return pl.pallas_call(
        _tc_gauss_second,
        grid=(nblk,),
        in_specs=[xspec, small, small,
                  pl.BlockSpec(memory_space=pltpu.MemorySpace.HBM)],
        out_specs=pl.BlockSpec((_RB, _N, _K), lambda i: (i + nblk, 0, 0)),
        out_shape=out_shape,
        input_output_aliases={3: 0},
    )(xe2, means, stds, buf)


@jax.jit
def kernel(x, edge_type, means, stds, mul, bias):
    x2 = x.reshape(_ROWS, _N)
    et2 = edge_type.reshape(_ROWS, _N)
    mulf = mul.reshape(_ET)
    biasf = bias.reshape(_ET)
    xe1 = _sc_affine(x2, et2, mulf, biasf, 0, _HALF)
    xe2 = _sc_affine(x2, et2, mulf, biasf, _HALF, _HALF)
    out = _tc_gauss(xe1, xe2, means, stds)
    return out.reshape(_B, _N, _N, _K)


# speedup vs baseline: 1.0694x; 1.0694x over previous
import functools
import math

import jax
import jax.numpy as jnp
from jax import lax
from jax.experimental import pallas as pl
from jax.experimental.pallas import tpu as pltpu
from jax.experimental.pallas import tpu_sc as plsc

_B, _N, _K, _ET = 4, 256, 128, 1024
_ROWS = _B * _N
_HALF = _ROWS // 2
_NC, _NS = 2, 16
_NW = _NC * _NS
_LANES = 16
_RB = 128


def _sc_affine(x2, et2, mul_flat, bias_flat, base, nrows):
    rpw = nrows // _NW
    nsl = _N // _LANES

    def body(x_hbm, et_hbm, mul_hbm, bias_hbm, out_hbm,
             xv, etv, mulv, biasv, outv):
        wid = lax.axis_index("s") * _NC + lax.axis_index("c")
        src = base + wid * rpw
        dst = wid * rpw
        pltpu.sync_copy(mul_hbm, mulv)
        pltpu.sync_copy(bias_hbm, biasv)
        pltpu.sync_copy(x_hbm.at[pl.ds(src, rpw)], xv)
        pltpu.sync_copy(et_hbm.at[pl.ds(src, rpw)], etv)

        @plsc.parallel_loop(0, rpw * nsl, unroll=8)
        def _(i):
            r = i // nsl
            sl = pl.ds((i % nsl) * _LANES, _LANES)
            idx = etv[r, sl]
            m = plsc.load_gather(mulv, [idx])
            b = plsc.load_gather(biasv, [idx])
            outv[r, sl] = m * xv[r, sl] + b

        pltpu.sync_copy(outv, out_hbm.at[pl.ds(dst, rpw)])

    mesh = plsc.VectorSubcoreMesh(core_axis_name="c", subcore_axis_name="s")
    kern = functools.partial(
        pl.kernel,
        mesh=mesh,
        compiler_params=pltpu.CompilerParams(needs_layout_passes=False),
        out_type=jax.ShapeDtypeStruct((nrows, _N), jnp.float32),
        scratch_types=[
            pltpu.VMEM((rpw, _N), jnp.float32),
            pltpu.VMEM((rpw, _N), jnp.int32),
            pltpu.VMEM((_ET,), jnp.float32),
            pltpu.VMEM((_ET,), jnp.float32),
            pltpu.VMEM((rpw, _N), jnp.float32),
        ],
    )(body)
    return kern(x2, et2, mul_flat, bias_flat)


def _gauss_body(xe_ref, mean_ref, std_ref, out_ref):
    mean = mean_ref[...].reshape(1, 1, _K)
    s = jnp.abs(std_ref[...]).reshape(1, 1, _K) + 1e-5
    g = math.sqrt(0.5 * math.log2(math.e)) / s
    log2coef = -jnp.log2(s) - math.log2(math.sqrt(2.0 * math.pi))
    xv = xe_ref[...]
    t = (xv[:, :, None] - mean) * g
    out_ref[...] = jnp.exp2(log2coef - t * t)


def _tc_gauss(xe, means, stds):
    nblk = _ROWS // _RB
    small = pl.BlockSpec((1, _K), lambda i: (0, 0))
    return pl.pallas_call(
        _gauss_body,
        grid=(nblk,),
        in_specs=[pl.BlockSpec((_RB, _N), lambda i: (i, 0)), small, small],
        out_specs=pl.BlockSpec((_RB, _N, _K), lambda i: (i, 0, 0)),
        out_shape=jax.ShapeDtypeStruct((_ROWS, _N, _K), jnp.float32),
    )(xe, means, stds)


@jax.jit
def kernel(x, edge_type, means, stds, mul, bias):
    x2 = x.reshape(_ROWS, _N)
    et2 = edge_type.reshape(_ROWS, _N)
    mulf = mul.reshape(_ET)
    biasf = bias.reshape(_ET)
    xe = _sc_affine(x2, et2, mulf, biasf, 0, _ROWS)
    out = _tc_gauss(xe, means, stds)
    return out.reshape(_B, _N, _N, _K)
